# baseline (speedup 1.0000x reference)
import functools

import jax
import jax.numpy as jnp
from jax import lax
from jax.experimental import pallas as pl
from jax.experimental.pallas import tpu as pltpu
from jax.experimental.pallas import tpu_sc as plsc

VOCAB = 1000
B, T = 1024, 50
N_ROWS = B * T

NC, NS, L = 2, 16, 16
NW = NC * NS
B_PER_W = B // NW
NBUF = 2
TPAD = 64


def _lse_body(table_ref, lse_ref):
    t = table_ref[...]
    m = jnp.max(t, axis=1, keepdims=True)
    s = jnp.sum(jnp.exp(t - m), axis=1, keepdims=True)
    lse_ref[...] = jnp.log(s) + m


def _lse_table(table):
    return pl.pallas_call(
        _lse_body,
        out_shape=jax.ShapeDtypeStruct((VOCAB, 1), jnp.float32),
    )(table)


def _loss_body(part_ref, out_ref):
    out_ref[...] = jnp.sum(part_ref[...], keepdims=True) * (1.0 / N_ROWS)


def _loss_reduce(partials):
    return pl.pallas_call(
        _loss_body,
        out_shape=jax.ShapeDtypeStruct((1, 1), jnp.float32),
    )(partials)


TBLK = 5


def _mm_body(xt_ref, tab_ref, out_ref):
    for i in range(TBLK):
        xv = xt_ref[i, 0, :]
        wiota = lax.broadcasted_iota(jnp.int32, (VOCAB, B), 0)
        oh = (wiota == xv[None, :]).astype(jnp.bfloat16)
        out_ref[i] = lax.dot_general(
            tab_ref[...], oh, (((1,), (0,)), ((), ())),
            preferred_element_type=jnp.float32)


def _logits_mm(x_t, table_bf):
    return pl.pallas_call(
        _mm_body,
        grid=(T // TBLK,),
        in_specs=[
            pl.BlockSpec((TBLK, 1, B), lambda t: (t, 0, 0)),
            pl.BlockSpec((VOCAB, VOCAB), lambda t: (0, 0)),
        ],
        out_specs=pl.BlockSpec((TBLK, VOCAB, B), lambda t: (t, 0, 0)),
        out_shape=jax.ShapeDtypeStruct((T, VOCAB, B), jnp.float32),
    )(x_t, table_bf)


ROWS_PER_W = N_ROWS // NW
GCHUNK = 128
NGROUP = ROWS_PER_W // L


def _sc_body(tab16_hbm, x_hbm, t_hbm, lse_hbm, part_hbm,
             xv_v, tv_v, row_v, lane_v, pick_v, lse_v, acc_v, sem):
    wid = lax.axis_index("s") * NC + lax.axis_index("c")
    base = wid * ROWS_PER_W

    pltpu.sync_copy(x_hbm.at[pl.ds(base, ROWS_PER_W)], xv_v)
    pltpu.sync_copy(t_hbm.at[pl.ds(base, ROWS_PER_W)], tv_v)
    pltpu.sync_copy(lse_hbm, lse_v)
    acc_v[...] = jnp.zeros((L,), jnp.float32)

    for j in range(NGROUP):
        s = pl.ds(j * L, L)
        f = xv_v[s] * VOCAB + tv_v[s]
        row_v[s] = lax.shift_right_logical(f, 4)
        lane_v[s] = lax.bitwise_and(f, 15)

    nchunk = ROWS_PER_W // GCHUNK
    for c in range(nchunk):
        pltpu.async_copy(
            tab16_hbm.at[row_v.at[pl.ds(c * GCHUNK, GCHUNK)]],
            pick_v.at[pl.ds(c * GCHUNK, GCHUNK)], sem)
    for c in range(nchunk):
        pltpu.make_async_copy(
            tab16_hbm.at[row_v.at[pl.ds(c * GCHUNK, GCHUNK)]],
            pick_v.at[pl.ds(c * GCHUNK, GCHUNK)], sem).wait()

    for j in range(NGROUP):
        s = pl.ds(j * L, L)
        lsev = plsc.load_gather(lse_v, [xv_v[s]])
        rowids = lax.iota(jnp.int32, L) + j * L
        picked = plsc.load_gather(pick_v, [rowids, lane_v[s]])
        acc_v[...] = acc_v[...] + (lsev - picked)

    pltpu.sync_copy(acc_v, part_hbm.at[wid])


@functools.lru_cache(maxsize=1)
def _sc_loss():
    return pl.kernel(
        _sc_body,
        out_type=jax.ShapeDtypeStruct((NW, L), jnp.float32),
        mesh=plsc.VectorSubcoreMesh(
            core_axis_name="c", subcore_axis_name="s", num_cores=NC,
            num_subcores=NS),
        scratch_types=(
            pltpu.VMEM((ROWS_PER_W,), jnp.int32),
            pltpu.VMEM((ROWS_PER_W,), jnp.int32),
            pltpu.VMEM((ROWS_PER_W,), jnp.int32),
            pltpu.VMEM((ROWS_PER_W,), jnp.int32),
            pltpu.VMEM((ROWS_PER_W, L), jnp.float32),
            pltpu.VMEM((VOCAB,), jnp.float32),
            pltpu.VMEM((L,), jnp.float32),
            pltpu.SemaphoreType.DMA,
        ),
        compiler_params=pltpu.CompilerParams(
            needs_layout_passes=False, use_tc_tiling_on_sc=False),
    )


def kernel(x, targets, next_token_table):
    xi = x.astype(jnp.int32)
    ti = targets.astype(jnp.int32)
    lse = _lse_table(next_token_table).reshape(VOCAB)
    partials = _sc_loss()(
        next_token_table.reshape(VOCAB * VOCAB // L, L),
        xi.reshape(N_ROWS), ti.reshape(N_ROWS), lse)
    logits_tvb = _logits_mm(
        xi.T.reshape(T, 1, B), next_token_table.T.astype(jnp.bfloat16))
    loss = _loss_reduce(partials)
    return jnp.transpose(logits_tvb, (2, 0, 1)), loss[0, 0]


# speedup vs baseline: 1.0128x; 1.0128x over previous
import functools

import jax
import jax.numpy as jnp
from jax import lax
from jax.experimental import pallas as pl
from jax.experimental.pallas import tpu as pltpu
from jax.experimental.pallas import tpu_sc as plsc

VOCAB = 1000
B, T = 1024, 50
N_ROWS = B * T

NC, NS, L = 2, 16, 16
NW = NC * NS
B_PER_W = B // NW
NBUF = 2
TPAD = 64


def _lse_body(table_ref, lse_ref):
    t = table_ref[...]
    m = jnp.max(t, axis=1, keepdims=True)
    s = jnp.sum(jnp.exp(t - m), axis=1, keepdims=True)
    lse_ref[...] = jnp.log(s) + m


def _lse_table(table):
    return pl.pallas_call(
        _lse_body,
        out_shape=jax.ShapeDtypeStruct((VOCAB, 1), jnp.float32),
    )(table)


def _loss_body(part_ref, out_ref):
    out_ref[...] = jnp.sum(part_ref[...], keepdims=True) * (1.0 / N_ROWS)


def _loss_reduce(partials):
    return pl.pallas_call(
        _loss_body,
        out_shape=jax.ShapeDtypeStruct((1, 1), jnp.float32),
    )(partials)


TBLK = 2


def _mm_body(xt_ref, tab_ref, out_ref):
    for i in range(TBLK):
        xv = xt_ref[i, 0, :]
        wiota = lax.broadcasted_iota(jnp.int32, (VOCAB, B), 0)
        oh = (wiota == xv[None, :]).astype(jnp.bfloat16)
        out_ref[i] = lax.dot_general(
            tab_ref[...], oh, (((1,), (0,)), ((), ())),
            preferred_element_type=jnp.float32)


def _logits_mm(x_t, table_bf):
    return pl.pallas_call(
        _mm_body,
        grid=(T // TBLK,),
        in_specs=[
            pl.BlockSpec((TBLK, 1, B), lambda t: (t, 0, 0)),
            pl.BlockSpec((VOCAB, VOCAB), lambda t: (0, 0)),
        ],
        out_specs=pl.BlockSpec((TBLK, VOCAB, B), lambda t: (t, 0, 0)),
        out_shape=jax.ShapeDtypeStruct((T, VOCAB, B), jnp.float32),
    )(x_t, table_bf)


ROWS_PER_W = N_ROWS // NW
GCHUNK = 128
NGROUP = ROWS_PER_W // L


def _sc_body(tab16_hbm, x_hbm, t_hbm, lse_hbm, part_hbm,
             xv_v, tv_v, row_v, lane_v, pick_v, lse_v, acc_v, sem):
    wid = lax.axis_index("s") * NC + lax.axis_index("c")
    base = wid * ROWS_PER_W

    pltpu.sync_copy(x_hbm.at[pl.ds(base, ROWS_PER_W)], xv_v)
    pltpu.sync_copy(t_hbm.at[pl.ds(base, ROWS_PER_W)], tv_v)
    pltpu.sync_copy(lse_hbm, lse_v)
    acc_v[...] = jnp.zeros((L,), jnp.float32)

    for j in range(NGROUP):
        s = pl.ds(j * L, L)
        f = xv_v[s] * VOCAB + tv_v[s]
        row_v[s] = lax.shift_right_logical(f, 4)
        lane_v[s] = lax.bitwise_and(f, 15)

    nchunk = ROWS_PER_W // GCHUNK
    for c in range(nchunk):
        pltpu.async_copy(
            tab16_hbm.at[row_v.at[pl.ds(c * GCHUNK, GCHUNK)]],
            pick_v.at[pl.ds(c * GCHUNK, GCHUNK)], sem)
    for c in range(nchunk):
        pltpu.make_async_copy(
            tab16_hbm.at[row_v.at[pl.ds(c * GCHUNK, GCHUNK)]],
            pick_v.at[pl.ds(c * GCHUNK, GCHUNK)], sem).wait()

    for j in range(NGROUP):
        s = pl.ds(j * L, L)
        lsev = plsc.load_gather(lse_v, [xv_v[s]])
        rowids = lax.iota(jnp.int32, L) + j * L
        picked = plsc.load_gather(pick_v, [rowids, lane_v[s]])
        acc_v[...] = acc_v[...] + (lsev - picked)

    pltpu.sync_copy(acc_v, part_hbm.at[wid])


@functools.lru_cache(maxsize=1)
def _sc_loss():
    return pl.kernel(
        _sc_body,
        out_type=jax.ShapeDtypeStruct((NW, L), jnp.float32),
        mesh=plsc.VectorSubcoreMesh(
            core_axis_name="c", subcore_axis_name="s", num_cores=NC,
            num_subcores=NS),
        scratch_types=(
            pltpu.VMEM((ROWS_PER_W,), jnp.int32),
            pltpu.VMEM((ROWS_PER_W,), jnp.int32),
            pltpu.VMEM((ROWS_PER_W,), jnp.int32),
            pltpu.VMEM((ROWS_PER_W,), jnp.int32),
            pltpu.VMEM((ROWS_PER_W, L), jnp.float32),
            pltpu.VMEM((VOCAB,), jnp.float32),
            pltpu.VMEM((L,), jnp.float32),
            pltpu.SemaphoreType.DMA,
        ),
        compiler_params=pltpu.CompilerParams(
            needs_layout_passes=False, use_tc_tiling_on_sc=False),
    )


def kernel(x, targets, next_token_table):
    xi = x.astype(jnp.int32)
    ti = targets.astype(jnp.int32)
    lse = _lse_table(next_token_table).reshape(VOCAB)
    xt = xi.T
    partials = _sc_loss()(
        next_token_table.reshape(VOCAB * VOCAB // L, L),
        xt.reshape(N_ROWS), ti.T.reshape(N_ROWS), lse)
    logits_tvb = _logits_mm(
        xt.reshape(T, 1, B), next_token_table.T.astype(jnp.bfloat16))
    loss = _loss_reduce(partials)
    return jnp.transpose(logits_tvb, (2, 0, 1)), loss[0, 0]


# speedup vs baseline: 1.0197x; 1.0069x over previous
import functools

import jax
import jax.numpy as jnp
from jax import lax
from jax.experimental import pallas as pl
from jax.experimental.pallas import tpu as pltpu
from jax.experimental.pallas import tpu_sc as plsc

VOCAB = 1000
B, T = 1024, 50
N_ROWS = B * T

NC, NS, L = 2, 16, 16
NW = NC * NS


def _lse_body(table_ref, lse_ref):
    t = table_ref[...]
    m = jnp.max(t, axis=1, keepdims=True)
    s = jnp.sum(jnp.exp(t - m), axis=1, keepdims=True)
    lse_ref[...] = jnp.log(s) + m


def _lse_table(table):
    return pl.pallas_call(
        _lse_body,
        out_shape=jax.ShapeDtypeStruct((VOCAB, 1), jnp.float32),
    )(table)


def _loss_body(part_ref, out_ref):
    out_ref[...] = jnp.sum(part_ref[...]).reshape(1, 1) * (1.0 / N_ROWS)


def _loss_reduce(partials):
    return pl.pallas_call(
        _loss_body,
        out_shape=jax.ShapeDtypeStruct((1, 1), jnp.float32),
    )(partials)


TBLK = 2


def _mm_body(xt_ref, tab_ref, out_ref):
    for i in range(TBLK):
        xv = xt_ref[i, 0, :]
        wiota = lax.broadcasted_iota(jnp.int32, (VOCAB, B), 0)
        oh = (wiota == xv[None, :]).astype(jnp.bfloat16)
        out_ref[i] = lax.dot_general(
            tab_ref[...], oh, (((1,), (0,)), ((), ())),
            preferred_element_type=jnp.float32)


def _logits_mm(x_t, table_bf):
    return pl.pallas_call(
        _mm_body,
        grid=(T // TBLK,),
        in_specs=[
            pl.BlockSpec((TBLK, 1, B), lambda t: (t, 0, 0)),
            pl.BlockSpec((VOCAB, VOCAB), lambda t: (0, 0)),
        ],
        out_specs=pl.BlockSpec((TBLK, VOCAB, B), lambda t: (t, 0, 0)),
        out_shape=jax.ShapeDtypeStruct((T, VOCAB, B), jnp.float32),
    )(x_t, table_bf)


ROWS_PER_W = N_ROWS // NW
GCHUNK = 128
NGROUP = ROWS_PER_W // L


def _sc_body(tab16_hbm, x_hbm, t_hbm, lse_hbm, part_hbm,
             xv_v, tv_v, row_v, lane_v, pick_v, lse_v, acc_v, sem):
    wid = lax.axis_index("s") * NC + lax.axis_index("c")
    base = wid * ROWS_PER_W

    pltpu.sync_copy(x_hbm.at[pl.ds(base, ROWS_PER_W)], xv_v)
    pltpu.sync_copy(t_hbm.at[pl.ds(base, ROWS_PER_W)], tv_v)
    pltpu.sync_copy(lse_hbm, lse_v)
    acc_v[...] = jnp.zeros((L,), jnp.float32)

    for j in range(NGROUP):
        s = pl.ds(j * L, L)
        f = xv_v[s] * VOCAB + tv_v[s]
        row_v[s] = lax.shift_right_logical(f, 4)
        lane_v[s] = lax.bitwise_and(f, 15)

    nchunk = ROWS_PER_W // GCHUNK
    for c in range(nchunk):
        pltpu.async_copy(
            tab16_hbm.at[row_v.at[pl.ds(c * GCHUNK, GCHUNK)]],
            pick_v.at[pl.ds(c * GCHUNK, GCHUNK)], sem)
    for c in range(nchunk):
        pltpu.make_async_copy(
            tab16_hbm.at[row_v.at[pl.ds(c * GCHUNK, GCHUNK)]],
            pick_v.at[pl.ds(c * GCHUNK, GCHUNK)], sem).wait()

    for j in range(NGROUP):
        s = pl.ds(j * L, L)
        lsev = plsc.load_gather(lse_v, [xv_v[s]])
        rowids = lax.iota(jnp.int32, L) + j * L
        picked = plsc.load_gather(pick_v, [rowids, lane_v[s]])
        acc_v[...] = acc_v[...] + (lsev - picked)

    pltpu.sync_copy(acc_v, part_hbm.at[pl.ds(wid * L, L)])


@functools.lru_cache(maxsize=1)
def _sc_loss():
    return pl.kernel(
        _sc_body,
        out_type=jax.ShapeDtypeStruct((NW * L,), jnp.float32),
        mesh=plsc.VectorSubcoreMesh(
            core_axis_name="c", subcore_axis_name="s", num_cores=NC,
            num_subcores=NS),
        scratch_types=(
            pltpu.VMEM((ROWS_PER_W,), jnp.int32),
            pltpu.VMEM((ROWS_PER_W,), jnp.int32),
            pltpu.VMEM((ROWS_PER_W,), jnp.int32),
            pltpu.VMEM((ROWS_PER_W,), jnp.int32),
            pltpu.VMEM((ROWS_PER_W, L), jnp.float32),
            pltpu.VMEM((VOCAB,), jnp.float32),
            pltpu.VMEM((L,), jnp.float32),
            pltpu.SemaphoreType.DMA,
        ),
        compiler_params=pltpu.CompilerParams(
            needs_layout_passes=False, use_tc_tiling_on_sc=False),
    )


def kernel(x, targets, next_token_table):
    xi = x.astype(jnp.int32)
    ti = targets.astype(jnp.int32)
    lse = _lse_table(next_token_table).reshape(VOCAB)
    xt = xi.T
    partials = _sc_loss()(
        next_token_table.reshape(VOCAB * VOCAB // L, L),
        xt.reshape(N_ROWS), ti.T.reshape(N_ROWS), lse)
    logits_tvb = _logits_mm(
        xt.reshape(T, 1, B), next_token_table.T.astype(jnp.bfloat16))
    loss = _loss_reduce(partials)
    return jnp.transpose(logits_tvb, (2, 0, 1)), loss[0, 0]
